# Initial kernel scaffold; baseline (speedup 1.0000x reference)
import jax
import jax.numpy as jnp
from jax.experimental import pallas as pl


def kernel(x, q_s, q_l, q_g, xq, i, j):
    raise NotImplementedError("write your pallas kernel here")


# speedup vs baseline: 72.6921x; 72.6921x over previous
import functools

import jax
import jax.numpy as jnp
import numpy as np
from jax import lax
from jax.experimental import pallas as pl
from jax.experimental.pallas import tpu as pltpu
from jax.experimental.pallas import tpu_sc as plsc


def _build_ctab_kernel(M, N, dim, x_ref, qlbd_ref, qsbd_ref, qgbd_ref,
                       qli_ref, qlj_ref, qsj_ref, qgj_ref, p_ref, pm_ref,
                       out_ref, gmax_ref):
    a = pl.program_id(0)
    BD = N * dim
    f32 = jnp.float32

    x = x_ref[...]
    dx = x[1:, :] - x[:-1, :]
    mask0 = dx == 0.0
    dxi = jnp.where(mask0, 0.0, 1.0 / jnp.where(mask0, 1.0, dx))

    qlbd = qlbd_ref[...]
    qsbd = qsbd_ref[...]
    qgbd = qgbd_ref[...]
    P = p_ref[...]
    PM = pm_ref[...]

    ql_mean = jax.lax.dot_general(qlbd, PM, (((1,), (0,)), ((), ())),
                                  preferred_element_type=f32)
    qs_mean = jax.lax.dot_general(qsbd, PM, (((1,), (0,)), ((), ())),
                                  preferred_element_type=f32)
    qg_mean = jax.lax.dot_general(qgbd, PM, (((1,), (0,)), ((), ())),
                                  preferred_element_type=f32)

    def bounds(mid):
        r1 = 2.0 * mid[0:1] - mid[1:2]
        r0 = 2.0 * r1 - mid[0:1]
        rp = 2.0 * mid[-1:] - mid[-2:-1]
        return jnp.concatenate([r0, r1, mid, rp], axis=0)

    mm_mid = jnp.concatenate([
        (ql_mean[0:1] - qs_mean) * dxi[0:1],
        (ql_mean[1:M] - ql_mean[0:M - 1]) * dxi[1:M],
        (qg_mean - ql_mean[M - 1:M]) * dxi[M:M + 1],
    ], axis=0)
    mm4 = bounds(mm_mid)
    mm_last = 2.0 * mm4[-1:] - mm4[-2:-1]
    mm_full = jnp.concatenate([mm4, mm_last], axis=0)
    mm_bd = jax.lax.dot_general(mm_full, P, (((1,), (0,)), ((), ())),
                                preferred_element_type=f32)

    rows = lax.broadcasted_iota(jnp.int32, (BD, dim), 0)
    cols = lax.broadcasted_iota(jnp.int32, (BD, dim), 1)
    psel = (rows == a * dim + cols).astype(f32)
    a_slice = jax.lax.dot_general(qlbd, psel, (((1,), (0,)), ((), ())),
                                  preferred_element_type=f32)
    a_bd = jax.lax.dot_general(a_slice, P, (((1,), (0,)), ((), ())),
                               preferred_element_type=f32)

    m_mid = jnp.concatenate([
        (qlbd[0:1] - qsbd) * dxi[0:1],
        (qlbd[1:M] - a_bd[0:M - 1]) * dxi[1:M],
        (qgbd - qlbd[M - 1:M]) * dxi[M:M + 1],
    ], axis=0)
    m4 = bounds(m_mid)
    e = (jnp.abs(mm_bd[1:M + 5] - m4) +
         0.5 * jnp.abs(mm_bd[1:M + 5] + m4))
    f12 = e[2:M + 4] + e[0:M + 2]
    gm = jnp.max(f12).reshape(1, 1)
    prev = jnp.where(a == 0, -jnp.inf, gmax_ref[...])
    gmax_ref[...] = jnp.maximum(prev, gm)

    @pl.when(a == pl.num_programs(0) - 1)
    def _():
        gmax = gmax_ref[0, 0]
        qli = qli_ref[...]
        qlj = qlj_ref[...]
        qsj = qsj_ref[...]
        qgj = qgj_ref[...]
        mij_mid = jnp.concatenate([
            (qlj[0:1] - qsj) * dxi[0:1],
            (qlj[1:M] - qli[0:M - 1]) * dxi[1:M],
            (qgj - qlj[M - 1:M]) * dxi[M:M + 1],
        ], axis=0)
        mij4 = bounds(mij_mid)
        mij_last = 2.0 * mij4[-1:] - mij4[-2:-1]
        mij = jnp.concatenate([mij4, mij_last], axis=0)

        e_ij = (jnp.abs(mm_full[1:M + 5] - mij[0:M + 4]) +
                0.5 * jnp.abs(mm_full[1:M + 5] + mij[0:M + 4]))
        f1 = e_ij[2:M + 4]
        f2 = e_ij[0:M + 2]
        f12_ij = f1 + f2
        msk = f12_ij > 1e-09 * gmax
        df = (f1 * mij[1:M + 3] + f2 * mij[2:M + 4]) / jnp.where(msk, f12_ij, 1.0)
        df = jnp.where(msk, df, 0.5 * (mij[3:M + 5] + mij[0:M + 2]))

        slope = mij[2:M + 3]
        y0 = jnp.concatenate([qsj, qli[0:M - 1], qlj[M - 1:M]], axis=0)
        d0 = df[0:M + 1]
        d1 = df[1:M + 2]
        hinv = dxi
        c0 = (d0 + d1 - 2.0 * slope) * hinv * hinv
        c1 = (3.0 * slope - 2.0 * d0 - d1) * hinv
        out_ref[...] = jnp.concatenate([c0, c1, d0, y0], axis=1)


def _sc_eval(ctab, xq, M, dim, Q):
    NC, NS, L = 2, 16, 16
    NW = NC * NS
    CHUNK = 2048
    qpw = Q // NW
    nchunks = qpw // CHUNK
    rows = M + 1

    mesh = plsc.VectorSubcoreMesh(core_axis_name="c", subcore_axis_name="s",
                                  num_cores=NC, num_subcores=NS)

    @functools.partial(
        pl.kernel,
        out_type=jax.ShapeDtypeStruct((Q * dim,), jnp.float32),
        mesh=mesh,
        compiler_params=pltpu.CompilerParams(needs_layout_passes=False),
        scratch_types=[
            pltpu.VMEM((rows * 4 * dim,), jnp.float32),
            pltpu.VMEM((CHUNK,), jnp.float32),
            pltpu.VMEM((CHUNK * dim,), jnp.float32),
        ],
    )
    def sc_eval(ctab_hbm, xq_hbm, y_hbm, tab_v, xq_v, y_v):
        wid = lax.axis_index("s") * NC + lax.axis_index("c")
        base = wid * qpw
        pltpu.sync_copy(ctab_hbm, tab_v)
        iota = lax.iota(jnp.int32, L)
        for c in range(nchunks):
            start = base + c * CHUNK
            pltpu.sync_copy(xq_hbm.at[pl.ds(start, CHUNK)], xq_v)

            def body(g, carry):
                xv = xq_v[pl.ds(g * L, L)]
                seg = jnp.clip(xv.astype(jnp.int32), 0, M)
                t = xv - seg.astype(jnp.float32)
                tbase = seg * (4 * dim)
                obase = (g * L + iota) * dim
                for d in range(dim):
                    c0 = plsc.load_gather(tab_v, [tbase + d])
                    c1 = plsc.load_gather(tab_v, [tbase + (dim + d)])
                    c2 = plsc.load_gather(tab_v, [tbase + (2 * dim + d)])
                    c3 = plsc.load_gather(tab_v, [tbase + (3 * dim + d)])
                    y = ((c0 * t + c1) * t + c2) * t + c3
                    plsc.store_scatter(y_v, [obase + d], y)
                return carry

            lax.fori_loop(0, CHUNK // L, body, 0)
            pltpu.sync_copy(y_v, y_hbm.at[pl.ds(start * dim, CHUNK * dim)])

    return sc_eval(ctab.reshape(-1), xq).reshape(Q, dim)


def kernel(x, q_s, q_l, q_g, xq, i, j):
    M, N, dim = q_l.shape
    Q = xq.shape[0]
    f32 = jnp.float32
    BD = N * dim

    x2 = x.astype(f32).reshape(M + 2, 1)
    qlbd = q_l.reshape(M, BD)
    qsbd = q_s.reshape(1, BD)
    qgbd = q_g.reshape(1, BD)
    qli = lax.dynamic_index_in_dim(q_l, i, axis=1, keepdims=False)
    qlj = lax.dynamic_index_in_dim(q_l, j, axis=1, keepdims=False)
    qsj = lax.dynamic_index_in_dim(q_s, j, axis=0, keepdims=True)
    qgj = lax.dynamic_index_in_dim(q_g, j, axis=0, keepdims=True)

    P = np.tile(np.eye(dim, dtype=np.float32), (1, N))
    PM = (np.tile(np.eye(dim, dtype=np.float32), (N, 1)) / N)

    full = lambda s: pl.BlockSpec(s, lambda a: (0,) * len(s))
    ctab = pl.pallas_call(
        functools.partial(_build_ctab_kernel, M, N, dim),
        grid=(N,),
        in_specs=[
            full((M + 2, 1)), full((M, BD)), full((1, BD)), full((1, BD)),
            full((M, dim)), full((M, dim)), full((1, dim)), full((1, dim)),
            full((dim, BD)), full((BD, dim)),
        ],
        out_specs=full((M + 1, 4 * dim)),
        out_shape=jax.ShapeDtypeStruct((M + 1, 4 * dim), f32),
        scratch_shapes=[pltpu.VMEM((1, 1), f32)],
        compiler_params=pltpu.CompilerParams(
            dimension_semantics=("arbitrary",)),
    )(x2, qlbd, qsbd, qgbd, qli, qlj, qsj, qgj, jnp.asarray(P), jnp.asarray(PM))

    return _sc_eval(ctab, xq.astype(f32), M, dim, Q)


# speedup vs baseline: 76.0406x; 1.0461x over previous
import functools

import jax
import jax.numpy as jnp
import numpy as np
from jax import lax
from jax.experimental import pallas as pl
from jax.experimental.pallas import tpu as pltpu
from jax.experimental.pallas import tpu_sc as plsc


def _build_ctab_kernel(M, N, dim, x_ref, qlbd_ref, qsbd_ref, qgbd_ref,
                       qli_ref, qlj_ref, qsj_ref, qgj_ref, p_ref, pm_ref,
                       out_ref, gmax_ref):
    a = pl.program_id(0)
    BD = N * dim
    f32 = jnp.float32

    x = x_ref[...]
    dx = x[1:, :] - x[:-1, :]
    mask0 = dx == 0.0
    dxi = jnp.where(mask0, 0.0, 1.0 / jnp.where(mask0, 1.0, dx))

    qlbd = qlbd_ref[...]
    qsbd = qsbd_ref[...]
    qgbd = qgbd_ref[...]
    P = p_ref[...]
    PM = pm_ref[...]

    ql_mean = jax.lax.dot_general(qlbd, PM, (((1,), (0,)), ((), ())),
                                  preferred_element_type=f32)
    qs_mean = jax.lax.dot_general(qsbd, PM, (((1,), (0,)), ((), ())),
                                  preferred_element_type=f32)
    qg_mean = jax.lax.dot_general(qgbd, PM, (((1,), (0,)), ((), ())),
                                  preferred_element_type=f32)

    def bounds(mid):
        r1 = 2.0 * mid[0:1] - mid[1:2]
        r0 = 2.0 * r1 - mid[0:1]
        rp = 2.0 * mid[-1:] - mid[-2:-1]
        return jnp.concatenate([r0, r1, mid, rp], axis=0)

    mm_mid = jnp.concatenate([
        (ql_mean[0:1] - qs_mean) * dxi[0:1],
        (ql_mean[1:M] - ql_mean[0:M - 1]) * dxi[1:M],
        (qg_mean - ql_mean[M - 1:M]) * dxi[M:M + 1],
    ], axis=0)
    mm4 = bounds(mm_mid)
    mm_last = 2.0 * mm4[-1:] - mm4[-2:-1]
    mm_full = jnp.concatenate([mm4, mm_last], axis=0)
    mm_bd = jax.lax.dot_general(mm_full, P, (((1,), (0,)), ((), ())),
                                preferred_element_type=f32)

    rows = lax.broadcasted_iota(jnp.int32, (BD, dim), 0)
    cols = lax.broadcasted_iota(jnp.int32, (BD, dim), 1)
    psel = (rows == a * dim + cols).astype(f32)
    a_slice = jax.lax.dot_general(qlbd, psel, (((1,), (0,)), ((), ())),
                                  preferred_element_type=f32)
    a_bd = jax.lax.dot_general(a_slice, P, (((1,), (0,)), ((), ())),
                               preferred_element_type=f32)

    m_mid = jnp.concatenate([
        (qlbd[0:1] - qsbd) * dxi[0:1],
        (qlbd[1:M] - a_bd[0:M - 1]) * dxi[1:M],
        (qgbd - qlbd[M - 1:M]) * dxi[M:M + 1],
    ], axis=0)
    m4 = bounds(m_mid)
    e = (jnp.abs(mm_bd[1:M + 5] - m4) +
         0.5 * jnp.abs(mm_bd[1:M + 5] + m4))
    f12 = e[2:M + 4] + e[0:M + 2]
    gm = jnp.max(f12).reshape(1, 1)
    prev = jnp.where(a == 0, -jnp.inf, gmax_ref[...])
    gmax_ref[...] = jnp.maximum(prev, gm)

    @pl.when(a == pl.num_programs(0) - 1)
    def _():
        gmax = gmax_ref[0, 0]
        qli = qli_ref[...]
        qlj = qlj_ref[...]
        qsj = qsj_ref[...]
        qgj = qgj_ref[...]
        mij_mid = jnp.concatenate([
            (qlj[0:1] - qsj) * dxi[0:1],
            (qlj[1:M] - qli[0:M - 1]) * dxi[1:M],
            (qgj - qlj[M - 1:M]) * dxi[M:M + 1],
        ], axis=0)
        mij4 = bounds(mij_mid)
        mij_last = 2.0 * mij4[-1:] - mij4[-2:-1]
        mij = jnp.concatenate([mij4, mij_last], axis=0)

        e_ij = (jnp.abs(mm_full[1:M + 5] - mij[0:M + 4]) +
                0.5 * jnp.abs(mm_full[1:M + 5] + mij[0:M + 4]))
        f1 = e_ij[2:M + 4]
        f2 = e_ij[0:M + 2]
        f12_ij = f1 + f2
        msk = f12_ij > 1e-09 * gmax
        df = (f1 * mij[1:M + 3] + f2 * mij[2:M + 4]) / jnp.where(msk, f12_ij, 1.0)
        df = jnp.where(msk, df, 0.5 * (mij[3:M + 5] + mij[0:M + 2]))

        slope = mij[2:M + 3]
        y0 = jnp.concatenate([qsj, qli[0:M - 1], qlj[M - 1:M]], axis=0)
        d0 = df[0:M + 1]
        d1 = df[1:M + 2]
        hinv = dxi
        c0 = (d0 + d1 - 2.0 * slope) * hinv * hinv
        c1 = (3.0 * slope - 2.0 * d0 - d1) * hinv
        out_ref[...] = jnp.concatenate([c0, c1, d0, y0], axis=1)


def _sc_eval(ctab, xq, M, dim, Q):
    NC, NS, L = 2, 16, 16
    NW = NC * NS
    CHUNK = 2048
    qpw = Q // NW
    nchunks = qpw // CHUNK
    rows = M + 1

    mesh = plsc.VectorSubcoreMesh(core_axis_name="c", subcore_axis_name="s",
                                  num_cores=NC, num_subcores=NS)

    @functools.partial(
        pl.kernel,
        out_type=jax.ShapeDtypeStruct((Q * dim,), jnp.float32),
        mesh=mesh,
        compiler_params=pltpu.CompilerParams(needs_layout_passes=False,
                                             disable_bounds_checks=True),
        scratch_types=[
            pltpu.VMEM((rows * 4 * dim,), jnp.float32),
            pltpu.VMEM((CHUNK,), jnp.float32),
            pltpu.VMEM((CHUNK * dim,), jnp.float32),
        ],
    )
    def sc_eval(ctab_hbm, xq_hbm, y_hbm, tab_v, xq_v, y_v):
        wid = lax.axis_index("s") * NC + lax.axis_index("c")
        base = wid * qpw
        pltpu.sync_copy(ctab_hbm, tab_v)
        iota = lax.iota(jnp.int32, L)
        for c in range(nchunks):
            start = base + c * CHUNK
            pltpu.sync_copy(xq_hbm.at[pl.ds(start, CHUNK)], xq_v)

            @plsc.parallel_loop(0, CHUNK // L, unroll=4)
            def body(g):
                xv = xq_v[pl.ds(g * L, L)]
                seg = jnp.clip(xv.astype(jnp.int32), 0, M)
                t = xv - seg.astype(jnp.float32)
                tbase = seg * (4 * dim)
                obase = (g * L + iota) * dim
                for d in range(dim):
                    c0 = plsc.load_gather(tab_v, [tbase + d])
                    c1 = plsc.load_gather(tab_v, [tbase + (dim + d)])
                    c2 = plsc.load_gather(tab_v, [tbase + (2 * dim + d)])
                    c3 = plsc.load_gather(tab_v, [tbase + (3 * dim + d)])
                    y = ((c0 * t + c1) * t + c2) * t + c3
                    plsc.store_scatter(y_v, [obase + d], y)
            pltpu.sync_copy(y_v, y_hbm.at[pl.ds(start * dim, CHUNK * dim)])

    return sc_eval(ctab.reshape(-1), xq).reshape(Q, dim)


def kernel(x, q_s, q_l, q_g, xq, i, j):
    M, N, dim = q_l.shape
    Q = xq.shape[0]
    f32 = jnp.float32
    BD = N * dim

    x2 = x.astype(f32).reshape(M + 2, 1)
    qlbd = q_l.reshape(M, BD)
    qsbd = q_s.reshape(1, BD)
    qgbd = q_g.reshape(1, BD)
    qli = lax.dynamic_index_in_dim(q_l, i, axis=1, keepdims=False)
    qlj = lax.dynamic_index_in_dim(q_l, j, axis=1, keepdims=False)
    qsj = lax.dynamic_index_in_dim(q_s, j, axis=0, keepdims=True)
    qgj = lax.dynamic_index_in_dim(q_g, j, axis=0, keepdims=True)

    P = np.tile(np.eye(dim, dtype=np.float32), (1, N))
    PM = (np.tile(np.eye(dim, dtype=np.float32), (N, 1)) / N)

    full = lambda s: pl.BlockSpec(s, lambda a: (0,) * len(s))
    ctab = pl.pallas_call(
        functools.partial(_build_ctab_kernel, M, N, dim),
        grid=(N,),
        in_specs=[
            full((M + 2, 1)), full((M, BD)), full((1, BD)), full((1, BD)),
            full((M, dim)), full((M, dim)), full((1, dim)), full((1, dim)),
            full((dim, BD)), full((BD, dim)),
        ],
        out_specs=full((M + 1, 4 * dim)),
        out_shape=jax.ShapeDtypeStruct((M + 1, 4 * dim), f32),
        scratch_shapes=[pltpu.VMEM((1, 1), f32)],
        compiler_params=pltpu.CompilerParams(
            dimension_semantics=("arbitrary",)),
    )(x2, qlbd, qsbd, qgbd, qli, qlj, qsj, qgj, jnp.asarray(P), jnp.asarray(PM))

    return _sc_eval(ctab, xq.astype(f32), M, dim, Q)


# speedup vs baseline: 76.5037x; 1.0061x over previous
import functools

import jax
import jax.numpy as jnp
import numpy as np
from jax import lax
from jax.experimental import pallas as pl
from jax.experimental.pallas import tpu as pltpu
from jax.experimental.pallas import tpu_sc as plsc


def _build_ctab_kernel(M, N, dim, x_ref, qlbd_ref, qsbd_ref, qgbd_ref,
                       qli_ref, qlj_ref, qsj_ref, qgj_ref, p_ref, pm_ref,
                       out_ref, gmax_ref):
    a = pl.program_id(0)
    BD = N * dim
    f32 = jnp.float32

    x = x_ref[...]
    dx = x[1:, :] - x[:-1, :]
    mask0 = dx == 0.0
    dxi = jnp.where(mask0, 0.0, 1.0 / jnp.where(mask0, 1.0, dx))

    qlbd = qlbd_ref[...]
    qsbd = qsbd_ref[...]
    qgbd = qgbd_ref[...]
    P = p_ref[...]
    PM = pm_ref[...]

    ql_mean = jax.lax.dot_general(qlbd, PM, (((1,), (0,)), ((), ())),
                                  preferred_element_type=f32)
    qs_mean = jax.lax.dot_general(qsbd, PM, (((1,), (0,)), ((), ())),
                                  preferred_element_type=f32)
    qg_mean = jax.lax.dot_general(qgbd, PM, (((1,), (0,)), ((), ())),
                                  preferred_element_type=f32)

    def bounds(mid):
        r1 = 2.0 * mid[0:1] - mid[1:2]
        r0 = 2.0 * r1 - mid[0:1]
        rp = 2.0 * mid[-1:] - mid[-2:-1]
        return jnp.concatenate([r0, r1, mid, rp], axis=0)

    mm_mid = jnp.concatenate([
        (ql_mean[0:1] - qs_mean) * dxi[0:1],
        (ql_mean[1:M] - ql_mean[0:M - 1]) * dxi[1:M],
        (qg_mean - ql_mean[M - 1:M]) * dxi[M:M + 1],
    ], axis=0)
    mm4 = bounds(mm_mid)
    mm_last = 2.0 * mm4[-1:] - mm4[-2:-1]
    mm_full = jnp.concatenate([mm4, mm_last], axis=0)
    mm_bd = jax.lax.dot_general(mm_full, P, (((1,), (0,)), ((), ())),
                                preferred_element_type=f32)

    rows = lax.broadcasted_iota(jnp.int32, (BD, dim), 0)
    cols = lax.broadcasted_iota(jnp.int32, (BD, dim), 1)
    psel = (rows == a * dim + cols).astype(f32)
    a_slice = jax.lax.dot_general(qlbd, psel, (((1,), (0,)), ((), ())),
                                  preferred_element_type=f32)
    a_bd = jax.lax.dot_general(a_slice, P, (((1,), (0,)), ((), ())),
                               preferred_element_type=f32)

    m_mid = jnp.concatenate([
        (qlbd[0:1] - qsbd) * dxi[0:1],
        (qlbd[1:M] - a_bd[0:M - 1]) * dxi[1:M],
        (qgbd - qlbd[M - 1:M]) * dxi[M:M + 1],
    ], axis=0)
    m4 = bounds(m_mid)
    e = (jnp.abs(mm_bd[1:M + 5] - m4) +
         0.5 * jnp.abs(mm_bd[1:M + 5] + m4))
    f12 = e[2:M + 4] + e[0:M + 2]
    gm = jnp.max(f12).reshape(1, 1)
    prev = jnp.where(a == 0, -jnp.inf, gmax_ref[...])
    gmax_ref[...] = jnp.maximum(prev, gm)

    @pl.when(a == pl.num_programs(0) - 1)
    def _():
        gmax = gmax_ref[0, 0]
        qli = qli_ref[...]
        qlj = qlj_ref[...]
        qsj = qsj_ref[...]
        qgj = qgj_ref[...]
        mij_mid = jnp.concatenate([
            (qlj[0:1] - qsj) * dxi[0:1],
            (qlj[1:M] - qli[0:M - 1]) * dxi[1:M],
            (qgj - qlj[M - 1:M]) * dxi[M:M + 1],
        ], axis=0)
        mij4 = bounds(mij_mid)
        mij_last = 2.0 * mij4[-1:] - mij4[-2:-1]
        mij = jnp.concatenate([mij4, mij_last], axis=0)

        e_ij = (jnp.abs(mm_full[1:M + 5] - mij[0:M + 4]) +
                0.5 * jnp.abs(mm_full[1:M + 5] + mij[0:M + 4]))
        f1 = e_ij[2:M + 4]
        f2 = e_ij[0:M + 2]
        f12_ij = f1 + f2
        msk = f12_ij > 1e-09 * gmax
        df = (f1 * mij[1:M + 3] + f2 * mij[2:M + 4]) / jnp.where(msk, f12_ij, 1.0)
        df = jnp.where(msk, df, 0.5 * (mij[3:M + 5] + mij[0:M + 2]))

        slope = mij[2:M + 3]
        y0 = jnp.concatenate([qsj, qli[0:M - 1], qlj[M - 1:M]], axis=0)
        d0 = df[0:M + 1]
        d1 = df[1:M + 2]
        hinv = dxi
        c0 = (d0 + d1 - 2.0 * slope) * hinv * hinv
        c1 = (3.0 * slope - 2.0 * d0 - d1) * hinv
        out_ref[...] = jnp.concatenate([c0, c1, d0, y0], axis=1)


def _sc_eval(ctab, xq, M, dim, Q):
    NC, NS, L = 2, 16, 16
    NW = NC * NS
    CHUNK = 2048
    qpw = Q // NW
    nchunks = qpw // CHUNK
    rows = M + 1

    mesh = plsc.VectorSubcoreMesh(core_axis_name="c", subcore_axis_name="s",
                                  num_cores=NC, num_subcores=NS)

    @functools.partial(
        pl.kernel,
        out_type=jax.ShapeDtypeStruct((Q * dim,), jnp.float32),
        mesh=mesh,
        compiler_params=pltpu.CompilerParams(needs_layout_passes=False,
                                             disable_bounds_checks=True),
        scratch_types=[
            pltpu.VMEM((rows * 4 * dim,), jnp.float32),
            pltpu.VMEM((CHUNK,), jnp.float32),
            pltpu.VMEM((CHUNK * dim,), jnp.float32),
        ],
    )
    def sc_eval(ctab_hbm, xq_hbm, y_hbm, tab_v, xq_v, y_v):
        wid = lax.axis_index("s") * NC + lax.axis_index("c")
        base = wid * qpw
        pltpu.sync_copy(ctab_hbm, tab_v)
        iota = lax.iota(jnp.int32, L)
        for c in range(nchunks):
            start = base + c * CHUNK
            pltpu.sync_copy(xq_hbm.at[pl.ds(start, CHUNK)], xq_v)

            @plsc.parallel_loop(0, CHUNK // L, unroll=2)
            def body(g):
                xv = xq_v[pl.ds(g * L, L)]
                seg = jnp.clip(xv.astype(jnp.int32), 0, M)
                t = xv - seg.astype(jnp.float32)
                tbase = seg * (4 * dim)
                obase = (g * L + iota) * dim
                ys = []
                for d in range(dim):
                    c0 = plsc.load_gather(tab_v, [tbase + d])
                    c1 = plsc.load_gather(tab_v, [tbase + (dim + d)])
                    c2 = plsc.load_gather(tab_v, [tbase + (2 * dim + d)])
                    c3 = plsc.load_gather(tab_v, [tbase + (3 * dim + d)])
                    ys.append(((c0 * t + c1) * t + c2) * t + c3)
                for d in range(dim):
                    plsc.store_scatter(y_v, [obase + d], ys[d])
            pltpu.sync_copy(y_v, y_hbm.at[pl.ds(start * dim, CHUNK * dim)])

    return sc_eval(ctab.reshape(-1), xq).reshape(Q, dim)


def kernel(x, q_s, q_l, q_g, xq, i, j):
    M, N, dim = q_l.shape
    Q = xq.shape[0]
    f32 = jnp.float32
    BD = N * dim

    x2 = x.astype(f32).reshape(M + 2, 1)
    qlbd = q_l.reshape(M, BD)
    qsbd = q_s.reshape(1, BD)
    qgbd = q_g.reshape(1, BD)
    qli = lax.dynamic_index_in_dim(q_l, i, axis=1, keepdims=False)
    qlj = lax.dynamic_index_in_dim(q_l, j, axis=1, keepdims=False)
    qsj = lax.dynamic_index_in_dim(q_s, j, axis=0, keepdims=True)
    qgj = lax.dynamic_index_in_dim(q_g, j, axis=0, keepdims=True)

    P = np.tile(np.eye(dim, dtype=np.float32), (1, N))
    PM = (np.tile(np.eye(dim, dtype=np.float32), (N, 1)) / N)

    full = lambda s: pl.BlockSpec(s, lambda a: (0,) * len(s))
    ctab = pl.pallas_call(
        functools.partial(_build_ctab_kernel, M, N, dim),
        grid=(N,),
        in_specs=[
            full((M + 2, 1)), full((M, BD)), full((1, BD)), full((1, BD)),
            full((M, dim)), full((M, dim)), full((1, dim)), full((1, dim)),
            full((dim, BD)), full((BD, dim)),
        ],
        out_specs=full((M + 1, 4 * dim)),
        out_shape=jax.ShapeDtypeStruct((M + 1, 4 * dim), f32),
        scratch_shapes=[pltpu.VMEM((1, 1), f32)],
        compiler_params=pltpu.CompilerParams(
            dimension_semantics=("arbitrary",)),
    )(x2, qlbd, qsbd, qgbd, qli, qlj, qsj, qgj, jnp.asarray(P), jnp.asarray(PM))

    return _sc_eval(ctab, xq.astype(f32), M, dim, Q)


# speedup vs baseline: 145.1030x; 1.8967x over previous
import functools

import jax
import jax.numpy as jnp
import numpy as np
from jax import lax
from jax.experimental import pallas as pl
from jax.experimental.pallas import tpu as pltpu
from jax.experimental.pallas import tpu_sc as plsc


def _build_ctab_kernel(M, N, dim, x_ref, qlbd_ref, qsbd_ref, qgbd_ref,
                       qli_ref, qlj_ref, qsj_ref, qgj_ref, p_ref, pm_ref,
                       out_ref, gmax_ref):
    a = pl.program_id(0)
    BD = N * dim
    f32 = jnp.float32

    x = x_ref[...]
    dx = x[1:, :] - x[:-1, :]
    mask0 = dx == 0.0
    dxi = jnp.where(mask0, 0.0, 1.0 / jnp.where(mask0, 1.0, dx))

    qlbd = qlbd_ref[...]
    qsbd = qsbd_ref[...]
    qgbd = qgbd_ref[...]
    P = p_ref[...]
    PM = pm_ref[...]

    ql_mean = jax.lax.dot_general(qlbd, PM, (((1,), (0,)), ((), ())),
                                  preferred_element_type=f32)
    qs_mean = jax.lax.dot_general(qsbd, PM, (((1,), (0,)), ((), ())),
                                  preferred_element_type=f32)
    qg_mean = jax.lax.dot_general(qgbd, PM, (((1,), (0,)), ((), ())),
                                  preferred_element_type=f32)

    def bounds(mid):
        r1 = 2.0 * mid[0:1] - mid[1:2]
        r0 = 2.0 * r1 - mid[0:1]
        rp = 2.0 * mid[-1:] - mid[-2:-1]
        return jnp.concatenate([r0, r1, mid, rp], axis=0)

    mm_mid = jnp.concatenate([
        (ql_mean[0:1] - qs_mean) * dxi[0:1],
        (ql_mean[1:M] - ql_mean[0:M - 1]) * dxi[1:M],
        (qg_mean - ql_mean[M - 1:M]) * dxi[M:M + 1],
    ], axis=0)
    mm4 = bounds(mm_mid)
    mm_last = 2.0 * mm4[-1:] - mm4[-2:-1]
    mm_full = jnp.concatenate([mm4, mm_last], axis=0)
    mm_bd = jax.lax.dot_general(mm_full, P, (((1,), (0,)), ((), ())),
                                preferred_element_type=f32)

    rows = lax.broadcasted_iota(jnp.int32, (BD, dim), 0)
    cols = lax.broadcasted_iota(jnp.int32, (BD, dim), 1)
    psel = (rows == a * dim + cols).astype(f32)
    a_slice = jax.lax.dot_general(qlbd, psel, (((1,), (0,)), ((), ())),
                                  preferred_element_type=f32)
    a_bd = jax.lax.dot_general(a_slice, P, (((1,), (0,)), ((), ())),
                               preferred_element_type=f32)

    m_mid = jnp.concatenate([
        (qlbd[0:1] - qsbd) * dxi[0:1],
        (qlbd[1:M] - a_bd[0:M - 1]) * dxi[1:M],
        (qgbd - qlbd[M - 1:M]) * dxi[M:M + 1],
    ], axis=0)
    m4 = bounds(m_mid)
    e = (jnp.abs(mm_bd[1:M + 5] - m4) +
         0.5 * jnp.abs(mm_bd[1:M + 5] + m4))
    f12 = e[2:M + 4] + e[0:M + 2]
    gm = jnp.max(f12).reshape(1, 1)
    prev = jnp.where(a == 0, -jnp.inf, gmax_ref[...])
    gmax_ref[...] = jnp.maximum(prev, gm)

    @pl.when(a == pl.num_programs(0) - 1)
    def _():
        gmax = gmax_ref[0, 0]
        qli = qli_ref[...]
        qlj = qlj_ref[...]
        qsj = qsj_ref[...]
        qgj = qgj_ref[...]
        mij_mid = jnp.concatenate([
            (qlj[0:1] - qsj) * dxi[0:1],
            (qlj[1:M] - qli[0:M - 1]) * dxi[1:M],
            (qgj - qlj[M - 1:M]) * dxi[M:M + 1],
        ], axis=0)
        mij4 = bounds(mij_mid)
        mij_last = 2.0 * mij4[-1:] - mij4[-2:-1]
        mij = jnp.concatenate([mij4, mij_last], axis=0)

        e_ij = (jnp.abs(mm_full[1:M + 5] - mij[0:M + 4]) +
                0.5 * jnp.abs(mm_full[1:M + 5] + mij[0:M + 4]))
        f1 = e_ij[2:M + 4]
        f2 = e_ij[0:M + 2]
        f12_ij = f1 + f2
        msk = f12_ij > 1e-09 * gmax
        df = (f1 * mij[1:M + 3] + f2 * mij[2:M + 4]) / jnp.where(msk, f12_ij, 1.0)
        df = jnp.where(msk, df, 0.5 * (mij[3:M + 5] + mij[0:M + 2]))

        slope = mij[2:M + 3]
        y0 = jnp.concatenate([qsj, qli[0:M - 1], qlj[M - 1:M]], axis=0)
        d0 = df[0:M + 1]
        d1 = df[1:M + 2]
        hinv = dxi
        c0 = (d0 + d1 - 2.0 * slope) * hinv * hinv
        c1 = (3.0 * slope - 2.0 * d0 - d1) * hinv
        out_ref[...] = jnp.concatenate([c0, c1, d0, y0], axis=1)


def _sc_eval(ctab, xq, M, dim, Q):
    NC, NS, L = 2, 16, 16
    NW = NC * NS
    CHUNK = 2048
    qpw = Q // NW
    nchunks = qpw // CHUNK
    rows = M + 1

    mesh = plsc.VectorSubcoreMesh(core_axis_name="c", subcore_axis_name="s",
                                  num_cores=NC, num_subcores=NS)

    @functools.partial(
        pl.kernel,
        out_type=jax.ShapeDtypeStruct((Q * dim,), jnp.float32),
        mesh=mesh,
        compiler_params=pltpu.CompilerParams(needs_layout_passes=False,
                                             disable_bounds_checks=True),
        scratch_types=[
            pltpu.VMEM((rows, 4 * dim), jnp.float32),
            pltpu.VMEM((CHUNK,), jnp.float32),
            pltpu.VMEM((CHUNK * dim,), jnp.float32),
        ],
    )
    def sc_eval(ctab_hbm, xq_hbm, y_hbm, tab_v, xq_v, y_v):
        wid = lax.axis_index("s") * NC + lax.axis_index("c")
        base = wid * qpw
        pltpu.sync_copy(ctab_hbm, tab_v)
        for c in range(nchunks):
            start = base + c * CHUNK
            pltpu.sync_copy(xq_hbm.at[pl.ds(start, CHUNK)], xq_v)

            @plsc.parallel_loop(0, CHUNK // L, unroll=2)
            def body(g):
                xv = xq_v[pl.ds(g * L, L)]
                seg = jnp.clip(xv.astype(jnp.int32), 0, M)
                t16 = xv - seg.astype(jnp.float32)
                for l in range(L):
                    s = seg[l]
                    t = jnp.broadcast_to(t16[l], (L,))
                    c0 = tab_v[s, 0:dim]
                    c1 = tab_v[s, dim:2 * dim]
                    c2 = tab_v[s, 2 * dim:3 * dim]
                    c3 = tab_v[s, 3 * dim:4 * dim]
                    y_v[pl.ds((g * L + l) * dim, dim)] = (
                        ((c0 * t + c1) * t + c2) * t + c3)

            pltpu.sync_copy(y_v, y_hbm.at[pl.ds(start * dim, CHUNK * dim)])

    return sc_eval(ctab, xq).reshape(Q, dim)


def kernel(x, q_s, q_l, q_g, xq, i, j):
    M, N, dim = q_l.shape
    Q = xq.shape[0]
    f32 = jnp.float32
    BD = N * dim

    x2 = x.astype(f32).reshape(M + 2, 1)
    qlbd = q_l.reshape(M, BD)
    qsbd = q_s.reshape(1, BD)
    qgbd = q_g.reshape(1, BD)
    qli = lax.dynamic_index_in_dim(q_l, i, axis=1, keepdims=False)
    qlj = lax.dynamic_index_in_dim(q_l, j, axis=1, keepdims=False)
    qsj = lax.dynamic_index_in_dim(q_s, j, axis=0, keepdims=True)
    qgj = lax.dynamic_index_in_dim(q_g, j, axis=0, keepdims=True)

    P = np.tile(np.eye(dim, dtype=np.float32), (1, N))
    PM = (np.tile(np.eye(dim, dtype=np.float32), (N, 1)) / N)

    full = lambda s: pl.BlockSpec(s, lambda a: (0,) * len(s))
    ctab = pl.pallas_call(
        functools.partial(_build_ctab_kernel, M, N, dim),
        grid=(N,),
        in_specs=[
            full((M + 2, 1)), full((M, BD)), full((1, BD)), full((1, BD)),
            full((M, dim)), full((M, dim)), full((1, dim)), full((1, dim)),
            full((dim, BD)), full((BD, dim)),
        ],
        out_specs=full((M + 1, 4 * dim)),
        out_shape=jax.ShapeDtypeStruct((M + 1, 4 * dim), f32),
        scratch_shapes=[pltpu.VMEM((1, 1), f32)],
        compiler_params=pltpu.CompilerParams(
            dimension_semantics=("arbitrary",)),
    )(x2, qlbd, qsbd, qgbd, qli, qlj, qsj, qgj, jnp.asarray(P), jnp.asarray(PM))

    return _sc_eval(ctab, xq.astype(f32), M, dim, Q)


# speedup vs baseline: 147.9173x; 1.0194x over previous
import functools

import jax
import jax.numpy as jnp
import numpy as np
from jax import lax
from jax.experimental import pallas as pl
from jax.experimental.pallas import tpu as pltpu
from jax.experimental.pallas import tpu_sc as plsc


def _bounds(mid):
    r1 = 2.0 * mid[0:1] - mid[1:2]
    r0 = 2.0 * r1 - mid[0:1]
    rp = 2.0 * mid[-1:] - mid[-2:-1]
    return jnp.concatenate([r0, r1, mid, rp], axis=0)


def _dxi_of(x):
    dx = x[1:, :] - x[:-1, :]
    mask0 = dx == 0.0
    return jnp.where(mask0, 0.0, 1.0 / jnp.where(mask0, 1.0, dx))


def _mid_rows(first, mids, last, dxi, M):
    return jnp.concatenate([
        first * dxi[0:1],
        mids * dxi[1:M],
        last * dxi[M:M + 1],
    ], axis=0)


def _prep_kernel(M, N, dim, x_ref, qlbd_ref, qsbd_ref, qgbd_ref, p_ref,
                 pm_ref, mmbd_ref, mm_ref, dxi_ref):
    f32 = jnp.float32
    dxi = _dxi_of(x_ref[...])
    qlbd = qlbd_ref[...]
    dn = (((1,), (0,)), ((), ()))
    ql_mean = lax.dot_general(qlbd, pm_ref[...], dn, preferred_element_type=f32)
    qs_mean = lax.dot_general(qsbd_ref[...], pm_ref[...], dn,
                              preferred_element_type=f32)
    qg_mean = lax.dot_general(qgbd_ref[...], pm_ref[...], dn,
                              preferred_element_type=f32)
    mm_mid = _mid_rows(ql_mean[0:1] - qs_mean,
                       ql_mean[1:M] - ql_mean[0:M - 1],
                       qg_mean - ql_mean[M - 1:M], dxi, M)
    mm4 = _bounds(mm_mid)
    mm_last = 2.0 * mm4[-1:] - mm4[-2:-1]
    mm_full = jnp.concatenate([mm4, mm_last], axis=0)
    mmbd_ref[...] = lax.dot_general(mm_full, p_ref[...], dn,
                                    preferred_element_type=f32)
    mm_ref[...] = mm_full
    dxi_ref[...] = dxi


def _gmax_kernel(M, N, dim, qlbd_ref, qsbd_ref, qgbd_ref, mmbd_ref, dxi_ref,
                 p_ref, out_ref, acc_ref):
    a = pl.program_id(0)
    BD = N * dim
    f32 = jnp.float32
    dn = (((1,), (0,)), ((), ()))
    qlbd = qlbd_ref[...]
    dxi = dxi_ref[...]
    rows = lax.broadcasted_iota(jnp.int32, (BD, dim), 0)
    cols = lax.broadcasted_iota(jnp.int32, (BD, dim), 1)
    psel = (rows == a * dim + cols).astype(f32)
    a_slice = lax.dot_general(qlbd, psel, dn, preferred_element_type=f32)
    a_bd = lax.dot_general(a_slice, p_ref[...], dn,
                           preferred_element_type=f32)
    m_mid = _mid_rows(qlbd[0:1] - qsbd_ref[...],
                      qlbd[1:M] - a_bd[0:M - 1],
                      qgbd_ref[...] - qlbd[M - 1:M], dxi, M)
    m4 = _bounds(m_mid)
    mmbd = mmbd_ref[...]
    e = (jnp.abs(mmbd[1:M + 5] - m4) +
         0.5 * jnp.abs(mmbd[1:M + 5] + m4))
    f12 = e[2:M + 4] + e[0:M + 2]
    gm = jnp.max(f12).reshape(1, 1)
    prev = jnp.where(a == 0, -jnp.inf, acc_ref[...])
    acc_ref[...] = jnp.maximum(prev, gm)

    @pl.when(a == pl.num_programs(0) - 1)
    def _():
        out_ref[...] = acc_ref[...]


def _table_kernel(M, N, dim, mm_ref, dxi_ref, gmax_ref, qli_ref, qlj_ref,
                  qsj_ref, qgj_ref, out_ref):
    gmax = gmax_ref[0, 0]
    dxi = dxi_ref[...]
    mm_full = mm_ref[...]
    qli = qli_ref[...]
    qlj = qlj_ref[...]
    qsj = qsj_ref[...]
    qgj = qgj_ref[...]
    mij_mid = _mid_rows(qlj[0:1] - qsj, qlj[1:M] - qli[0:M - 1],
                        qgj - qlj[M - 1:M], dxi, M)
    mij4 = _bounds(mij_mid)
    mij_last = 2.0 * mij4[-1:] - mij4[-2:-1]
    mij = jnp.concatenate([mij4, mij_last], axis=0)

    e_ij = (jnp.abs(mm_full[1:M + 5] - mij[0:M + 4]) +
            0.5 * jnp.abs(mm_full[1:M + 5] + mij[0:M + 4]))
    f1 = e_ij[2:M + 4]
    f2 = e_ij[0:M + 2]
    f12_ij = f1 + f2
    msk = f12_ij > 1e-09 * gmax
    df = (f1 * mij[1:M + 3] + f2 * mij[2:M + 4]) / jnp.where(msk, f12_ij, 1.0)
    df = jnp.where(msk, df, 0.5 * (mij[3:M + 5] + mij[0:M + 2]))

    slope = mij[2:M + 3]
    y0 = jnp.concatenate([qsj, qli[0:M - 1], qlj[M - 1:M]], axis=0)
    d0 = df[0:M + 1]
    d1 = df[1:M + 2]
    hinv = dxi
    c0 = (d0 + d1 - 2.0 * slope) * hinv * hinv
    c1 = (3.0 * slope - 2.0 * d0 - d1) * hinv
    out_ref[...] = jnp.concatenate([c0, c1, d0, y0], axis=1)


def _sc_eval(ctab, xq, M, dim, Q):
    NC, NS, L = 2, 16, 16
    NW = NC * NS
    CHUNK = 2048
    qpw = Q // NW
    nchunks = qpw // CHUNK
    rows = M + 1

    mesh = plsc.VectorSubcoreMesh(core_axis_name="c", subcore_axis_name="s",
                                  num_cores=NC, num_subcores=NS)

    @functools.partial(
        pl.kernel,
        out_type=jax.ShapeDtypeStruct((Q * dim,), jnp.float32),
        mesh=mesh,
        compiler_params=pltpu.CompilerParams(needs_layout_passes=False,
                                             disable_bounds_checks=True),
        scratch_types=[
            pltpu.VMEM((rows, 4 * dim), jnp.float32),
            pltpu.VMEM((2, CHUNK), jnp.float32),
            pltpu.VMEM((CHUNK * dim,), jnp.float32),
            pltpu.SemaphoreType.DMA,
            pltpu.SemaphoreType.DMA,
        ],
    )
    def sc_eval(ctab_hbm, xq_hbm, y_hbm, tab_v, xq_v, y_v, si0, si1):
        wid = lax.axis_index("s") * NC + lax.axis_index("c")
        base = wid * qpw
        pltpu.sync_copy(ctab_hbm, tab_v)
        sin = (si0, si1)
        in_cp = [None, None]
        in_cp[0] = pltpu.async_copy(
            xq_hbm.at[pl.ds(base, CHUNK)], xq_v.at[0], sin[0])
        for c in range(nchunks):
            cur = c % 2
            start = base + c * CHUNK
            if c + 1 < nchunks:
                in_cp[1 - cur] = pltpu.async_copy(
                    xq_hbm.at[pl.ds(start + CHUNK, CHUNK)],
                    xq_v.at[1 - cur], sin[1 - cur])
            in_cp[cur].wait()

            @plsc.parallel_loop(0, CHUNK // L, unroll=4)
            def body(g):
                xv = xq_v[cur, pl.ds(g * L, L)]
                seg = jnp.clip(xv.astype(jnp.int32), 0, M)
                t16 = xv - seg.astype(jnp.float32)
                for l in range(L):
                    s = seg[l]
                    t = jnp.broadcast_to(t16[l], (L,))
                    c0 = tab_v[s, 0:dim]
                    c1 = tab_v[s, dim:2 * dim]
                    c2 = tab_v[s, 2 * dim:3 * dim]
                    c3 = tab_v[s, 3 * dim:4 * dim]
                    y_v[pl.ds((g * L + l) * dim, dim)] = (
                        ((c0 * t + c1) * t + c2) * t + c3)

            pltpu.sync_copy(y_v, y_hbm.at[pl.ds(start * dim, CHUNK * dim)])

    return sc_eval(ctab, xq).reshape(Q, dim)


def kernel(x, q_s, q_l, q_g, xq, i, j):
    M, N, dim = q_l.shape
    Q = xq.shape[0]
    f32 = jnp.float32
    BD = N * dim

    x2 = x.astype(f32).reshape(M + 2, 1)
    qlbd = q_l.reshape(M, BD)
    qsbd = q_s.reshape(1, BD)
    qgbd = q_g.reshape(1, BD)
    qli = lax.dynamic_index_in_dim(q_l, i, axis=1, keepdims=False)
    qlj = lax.dynamic_index_in_dim(q_l, j, axis=1, keepdims=False)
    qsj = lax.dynamic_index_in_dim(q_s, j, axis=0, keepdims=True)
    qgj = lax.dynamic_index_in_dim(q_g, j, axis=0, keepdims=True)

    P = jnp.asarray(np.tile(np.eye(dim, dtype=np.float32), (1, N)))
    PM = jnp.asarray(np.tile(np.eye(dim, dtype=np.float32), (N, 1)) / N)

    full = lambda s: pl.BlockSpec(s, lambda *_: (0,) * len(s))

    mm_bd, mm_full, dxi = pl.pallas_call(
        functools.partial(_prep_kernel, M, N, dim),
        in_specs=[full((M + 2, 1)), full((M, BD)), full((1, BD)),
                  full((1, BD)), full((dim, BD)), full((BD, dim))],
        out_specs=[full((M + 5, BD)), full((M + 5, dim)), full((M + 1, 1))],
        out_shape=[jax.ShapeDtypeStruct((M + 5, BD), f32),
                   jax.ShapeDtypeStruct((M + 5, dim), f32),
                   jax.ShapeDtypeStruct((M + 1, 1), f32)],
    )(x2, qlbd, qsbd, qgbd, P, PM)

    gmax = pl.pallas_call(
        functools.partial(_gmax_kernel, M, N, dim),
        grid=(N,),
        in_specs=[full((M, BD)), full((1, BD)), full((1, BD)),
                  full((M + 5, BD)), full((M + 1, 1)), full((dim, BD))],
        out_specs=full((1, 1)),
        out_shape=jax.ShapeDtypeStruct((1, 1), f32),
        scratch_shapes=[pltpu.VMEM((1, 1), f32)],
        compiler_params=pltpu.CompilerParams(
            dimension_semantics=("arbitrary",)),
    )(qlbd, qsbd, qgbd, mm_bd, dxi, P)

    ctab = pl.pallas_call(
        functools.partial(_table_kernel, M, N, dim),
        in_specs=[full((M + 5, dim)), full((M + 1, 1)), full((1, 1)),
                  full((M, dim)), full((M, dim)), full((1, dim)),
                  full((1, dim))],
        out_specs=full((M + 1, 4 * dim)),
        out_shape=jax.ShapeDtypeStruct((M + 1, 4 * dim), f32),
    )(mm_full, dxi, gmax, qli, qlj, qsj, qgj)

    return _sc_eval(ctab, xq.astype(f32), M, dim, Q)


# speedup vs baseline: 211.2873x; 1.4284x over previous
import functools

import jax
import jax.numpy as jnp
import numpy as np
from jax import lax
from jax.experimental import pallas as pl
from jax.experimental.pallas import tpu as pltpu
from jax.experimental.pallas import tpu_sc as plsc


def _bounds(mid):
    r1 = 2.0 * mid[0:1] - mid[1:2]
    r0 = 2.0 * r1 - mid[0:1]
    rp = 2.0 * mid[-1:] - mid[-2:-1]
    return jnp.concatenate([r0, r1, mid, rp], axis=0)


def _dxi_of(x):
    dx = x[1:, :] - x[:-1, :]
    mask0 = dx == 0.0
    return jnp.where(mask0, 0.0, 1.0 / jnp.where(mask0, 1.0, dx))


def _mid_rows(first, mids, last, dxi, M):
    return jnp.concatenate([
        first * dxi[0:1],
        mids * dxi[1:M],
        last * dxi[M:M + 1],
    ], axis=0)


def _prep_kernel(M, N, dim, x_ref, qlbd_ref, qsbd_ref, qgbd_ref, p_ref,
                 pm_ref, mmbd_ref, mm_ref, dxi_ref):
    f32 = jnp.float32
    dxi = _dxi_of(x_ref[...])
    qlbd = qlbd_ref[...]
    dn = (((1,), (0,)), ((), ()))
    ql_mean = lax.dot_general(qlbd, pm_ref[...], dn, preferred_element_type=f32)
    qs_mean = lax.dot_general(qsbd_ref[...], pm_ref[...], dn,
                              preferred_element_type=f32)
    qg_mean = lax.dot_general(qgbd_ref[...], pm_ref[...], dn,
                              preferred_element_type=f32)
    mm_mid = _mid_rows(ql_mean[0:1] - qs_mean,
                       ql_mean[1:M] - ql_mean[0:M - 1],
                       qg_mean - ql_mean[M - 1:M], dxi, M)
    mm4 = _bounds(mm_mid)
    mm_last = 2.0 * mm4[-1:] - mm4[-2:-1]
    mm_full = jnp.concatenate([mm4, mm_last], axis=0)
    mmbd_ref[...] = lax.dot_general(mm_full, p_ref[...], dn,
                                    preferred_element_type=f32)
    mm_ref[...] = mm_full
    dxi_ref[...] = dxi


def _gmax_kernel(M, N, dim, qlbd_ref, qsbd_ref, qgbd_ref, mmbd_ref, dxi_ref,
                 p_ref, out_ref, acc_ref):
    a = pl.program_id(0)
    BD = N * dim
    f32 = jnp.float32
    dn = (((1,), (0,)), ((), ()))
    qlbd = qlbd_ref[...]
    dxi = dxi_ref[...]
    rows = lax.broadcasted_iota(jnp.int32, (BD, dim), 0)
    cols = lax.broadcasted_iota(jnp.int32, (BD, dim), 1)
    psel = (rows == a * dim + cols).astype(f32)
    a_slice = lax.dot_general(qlbd, psel, dn, preferred_element_type=f32)
    a_bd = lax.dot_general(a_slice, p_ref[...], dn,
                           preferred_element_type=f32)
    m_mid = _mid_rows(qlbd[0:1] - qsbd_ref[...],
                      qlbd[1:M] - a_bd[0:M - 1],
                      qgbd_ref[...] - qlbd[M - 1:M], dxi, M)
    m4 = _bounds(m_mid)
    mmbd = mmbd_ref[...]
    e = (jnp.abs(mmbd[1:M + 5] - m4) +
         0.5 * jnp.abs(mmbd[1:M + 5] + m4))
    f12 = e[2:M + 4] + e[0:M + 2]
    gm = jnp.max(f12).reshape(1, 1)
    prev = jnp.where(a == 0, -jnp.inf, acc_ref[...])
    acc_ref[...] = jnp.maximum(prev, gm)

    @pl.when(a == pl.num_programs(0) - 1)
    def _():
        out_ref[...] = acc_ref[...]


def _table_kernel(M, N, dim, mm_ref, dxi_ref, gmax_ref, qli_ref, qlj_ref,
                  qsj_ref, qgj_ref, out_ref):
    gmax = gmax_ref[0, 0]
    dxi = dxi_ref[...]
    mm_full = mm_ref[...]
    qli = qli_ref[...]
    qlj = qlj_ref[...]
    qsj = qsj_ref[...]
    qgj = qgj_ref[...]
    mij_mid = _mid_rows(qlj[0:1] - qsj, qlj[1:M] - qli[0:M - 1],
                        qgj - qlj[M - 1:M], dxi, M)
    mij4 = _bounds(mij_mid)
    mij_last = 2.0 * mij4[-1:] - mij4[-2:-1]
    mij = jnp.concatenate([mij4, mij_last], axis=0)

    e_ij = (jnp.abs(mm_full[1:M + 5] - mij[0:M + 4]) +
            0.5 * jnp.abs(mm_full[1:M + 5] + mij[0:M + 4]))
    f1 = e_ij[2:M + 4]
    f2 = e_ij[0:M + 2]
    f12_ij = f1 + f2
    msk = f12_ij > 1e-09 * gmax
    df = (f1 * mij[1:M + 3] + f2 * mij[2:M + 4]) / jnp.where(msk, f12_ij, 1.0)
    df = jnp.where(msk, df, 0.5 * (mij[3:M + 5] + mij[0:M + 2]))

    slope = mij[2:M + 3]
    y0 = jnp.concatenate([qsj, qli[0:M - 1], qlj[M - 1:M]], axis=0)
    d0 = df[0:M + 1]
    d1 = df[1:M + 2]
    hinv = dxi
    c0 = (d0 + d1 - 2.0 * slope) * hinv * hinv
    c1 = (3.0 * slope - 2.0 * d0 - d1) * hinv
    out_ref[...] = jnp.concatenate([c0, c1, d0, y0], axis=1)


def _sc_eval(ctab, xq, M, dim, Q):
    NC, NS, L = 2, 16, 16
    NW = NC * NS
    CHUNK = 2048
    qpw = Q // NW
    nchunks = qpw // CHUNK
    rows = M + 1

    mesh = plsc.VectorSubcoreMesh(core_axis_name="c", subcore_axis_name="s",
                                  num_cores=NC, num_subcores=NS)

    @functools.partial(
        pl.kernel,
        out_type=jax.ShapeDtypeStruct((dim, Q), jnp.float32),
        mesh=mesh,
        compiler_params=pltpu.CompilerParams(needs_layout_passes=False,
                                             disable_bounds_checks=True),
        scratch_types=[
            pltpu.VMEM((4 * dim, rows), jnp.float32),
            pltpu.VMEM((2, CHUNK), jnp.float32),
            pltpu.VMEM((dim, CHUNK), jnp.float32),
            pltpu.SemaphoreType.DMA,
            pltpu.SemaphoreType.DMA,
            pltpu.SemaphoreType.DMA,
        ],
    )
    def sc_eval(ctabt_hbm, xq_hbm, y_hbm, tab_v, xq_v, y_v, si0, si1, so):
        wid = lax.axis_index("s") * NC + lax.axis_index("c")
        base = wid * qpw
        pltpu.sync_copy(ctabt_hbm, tab_v)
        sin = (si0, si1)
        in_cp = [None, None]
        in_cp[0] = pltpu.async_copy(
            xq_hbm.at[pl.ds(base, CHUNK)], xq_v.at[0], sin[0])
        for c in range(nchunks):
            cur = c % 2
            start = base + c * CHUNK
            if c + 1 < nchunks:
                in_cp[1 - cur] = pltpu.async_copy(
                    xq_hbm.at[pl.ds(start + CHUNK, CHUNK)],
                    xq_v.at[1 - cur], sin[1 - cur])
            in_cp[cur].wait()
            if c > 0:
                for cp in out_cps:
                    cp.wait()

            @plsc.parallel_loop(0, CHUNK // L, unroll=2)
            def body(g):
                xv = xq_v[cur, pl.ds(g * L, L)]
                seg = jnp.clip(xv.astype(jnp.int32), 0, M)
                t = xv - seg.astype(jnp.float32)
                for d in range(dim):
                    k0 = jnp.full((L,), d, jnp.int32)
                    c0 = plsc.load_gather(tab_v, [k0, seg])
                    c1 = plsc.load_gather(tab_v, [k0 + dim, seg])
                    c2 = plsc.load_gather(tab_v, [k0 + 2 * dim, seg])
                    c3 = plsc.load_gather(tab_v, [k0 + 3 * dim, seg])
                    y_v[d, pl.ds(g * L, L)] = ((c0 * t + c1) * t + c2) * t + c3

            out_cps = [
                pltpu.async_copy(y_v.at[d], y_hbm.at[d, pl.ds(start, CHUNK)],
                                 so)
                for d in range(dim)
            ]
        for cp in out_cps:
            cp.wait()

    yt = sc_eval(jnp.transpose(ctab), xq)
    return jnp.transpose(yt)


def kernel(x, q_s, q_l, q_g, xq, i, j):
    M, N, dim = q_l.shape
    Q = xq.shape[0]
    f32 = jnp.float32
    BD = N * dim

    x2 = x.astype(f32).reshape(M + 2, 1)
    qlbd = q_l.reshape(M, BD)
    qsbd = q_s.reshape(1, BD)
    qgbd = q_g.reshape(1, BD)
    qli = lax.dynamic_index_in_dim(q_l, i, axis=1, keepdims=False)
    qlj = lax.dynamic_index_in_dim(q_l, j, axis=1, keepdims=False)
    qsj = lax.dynamic_index_in_dim(q_s, j, axis=0, keepdims=True)
    qgj = lax.dynamic_index_in_dim(q_g, j, axis=0, keepdims=True)

    P = jnp.asarray(np.tile(np.eye(dim, dtype=np.float32), (1, N)))
    PM = jnp.asarray(np.tile(np.eye(dim, dtype=np.float32), (N, 1)) / N)

    full = lambda s: pl.BlockSpec(s, lambda *_: (0,) * len(s))

    mm_bd, mm_full, dxi = pl.pallas_call(
        functools.partial(_prep_kernel, M, N, dim),
        in_specs=[full((M + 2, 1)), full((M, BD)), full((1, BD)),
                  full((1, BD)), full((dim, BD)), full((BD, dim))],
        out_specs=[full((M + 5, BD)), full((M + 5, dim)), full((M + 1, 1))],
        out_shape=[jax.ShapeDtypeStruct((M + 5, BD), f32),
                   jax.ShapeDtypeStruct((M + 5, dim), f32),
                   jax.ShapeDtypeStruct((M + 1, 1), f32)],
    )(x2, qlbd, qsbd, qgbd, P, PM)

    gmax = pl.pallas_call(
        functools.partial(_gmax_kernel, M, N, dim),
        grid=(N,),
        in_specs=[full((M, BD)), full((1, BD)), full((1, BD)),
                  full((M + 5, BD)), full((M + 1, 1)), full((dim, BD))],
        out_specs=full((1, 1)),
        out_shape=jax.ShapeDtypeStruct((1, 1), f32),
        scratch_shapes=[pltpu.VMEM((1, 1), f32)],
        compiler_params=pltpu.CompilerParams(
            dimension_semantics=("arbitrary",)),
    )(qlbd, qsbd, qgbd, mm_bd, dxi, P)

    ctab = pl.pallas_call(
        functools.partial(_table_kernel, M, N, dim),
        in_specs=[full((M + 5, dim)), full((M + 1, 1)), full((1, 1)),
                  full((M, dim)), full((M, dim)), full((1, dim)),
                  full((1, dim))],
        out_specs=full((M + 1, 4 * dim)),
        out_shape=jax.ShapeDtypeStruct((M + 1, 4 * dim), f32),
    )(mm_full, dxi, gmax, qli, qlj, qsj, qgj)

    return _sc_eval(ctab, xq.astype(f32), M, dim, Q)


# speedup vs baseline: 223.1990x; 1.0564x over previous
import functools

import jax
import jax.numpy as jnp
import numpy as np
from jax import lax
from jax.experimental import pallas as pl
from jax.experimental.pallas import tpu as pltpu
from jax.experimental.pallas import tpu_sc as plsc


def _bounds(mid):
    r1 = 2.0 * mid[0:1] - mid[1:2]
    r0 = 2.0 * r1 - mid[0:1]
    rp = 2.0 * mid[-1:] - mid[-2:-1]
    return jnp.concatenate([r0, r1, mid, rp], axis=0)


def _dxi_of(x):
    dx = x[1:, :] - x[:-1, :]
    mask0 = dx == 0.0
    return jnp.where(mask0, 0.0, 1.0 / jnp.where(mask0, 1.0, dx))


def _mid_rows(first, mids, last, dxi, M):
    return jnp.concatenate([
        first * dxi[0:1],
        mids * dxi[1:M],
        last * dxi[M:M + 1],
    ], axis=0)


def _prep_kernel(M, N, dim, x_ref, qlbd_ref, qsbd_ref, qgbd_ref, p_ref,
                 pm_ref, mmbd_ref, mm_ref, dxi_ref):
    f32 = jnp.float32
    dxi = _dxi_of(x_ref[...])
    qlbd = qlbd_ref[...]
    dn = (((1,), (0,)), ((), ()))
    ql_mean = lax.dot_general(qlbd, pm_ref[...], dn, preferred_element_type=f32)
    qs_mean = lax.dot_general(qsbd_ref[...], pm_ref[...], dn,
                              preferred_element_type=f32)
    qg_mean = lax.dot_general(qgbd_ref[...], pm_ref[...], dn,
                              preferred_element_type=f32)
    mm_mid = _mid_rows(ql_mean[0:1] - qs_mean,
                       ql_mean[1:M] - ql_mean[0:M - 1],
                       qg_mean - ql_mean[M - 1:M], dxi, M)
    mm4 = _bounds(mm_mid)
    mm_last = 2.0 * mm4[-1:] - mm4[-2:-1]
    mm_full = jnp.concatenate([mm4, mm_last], axis=0)
    mmbd_ref[...] = lax.dot_general(mm_full, p_ref[...], dn,
                                    preferred_element_type=f32)
    mm_ref[...] = mm_full
    dxi_ref[...] = dxi


def _gmax_kernel(M, N, dim, qlbd_ref, qsbd_ref, qgbd_ref, mmbd_ref, dxi_ref,
                 p_ref, out_ref, acc_ref):
    step = pl.program_id(0)
    BD = N * dim
    f32 = jnp.float32
    dn = (((1,), (0,)), ((), ()))
    qlbd = qlbd_ref[...]
    dxi = dxi_ref[...]
    mmbd = mmbd_ref[...]
    qsbd = qsbd_ref[...]
    qgbd = qgbd_ref[...]
    rows = lax.broadcasted_iota(jnp.int32, (BD, dim), 0)
    cols = lax.broadcasted_iota(jnp.int32, (BD, dim), 1)

    gm = None
    for half in range(2):
        a = step * 2 + half
        psel = (rows == a * dim + cols).astype(f32)
        a_slice = lax.dot_general(qlbd, psel, dn, preferred_element_type=f32)
        a_bd = lax.dot_general(a_slice, p_ref[...], dn,
                               preferred_element_type=f32)
        m_mid = _mid_rows(qlbd[0:1] - qsbd,
                          qlbd[1:M] - a_bd[0:M - 1],
                          qgbd - qlbd[M - 1:M], dxi, M)
        m4 = _bounds(m_mid)
        e = (jnp.abs(mmbd[1:M + 5] - m4) +
             0.5 * jnp.abs(mmbd[1:M + 5] + m4))
        f12 = e[2:M + 4] + e[0:M + 2]
        gm_h = jnp.max(f12).reshape(1, 1)
        gm = gm_h if gm is None else jnp.maximum(gm, gm_h)

    prev = jnp.where(step == 0, -jnp.inf, acc_ref[...])
    acc_ref[...] = jnp.maximum(prev, gm)

    @pl.when(step == pl.num_programs(0) - 1)
    def _():
        out_ref[...] = acc_ref[...]


def _table_kernel(M, N, dim, mm_ref, dxi_ref, gmax_ref, qli_ref, qlj_ref,
                  qsj_ref, qgj_ref, out_ref):
    gmax = gmax_ref[0, 0]
    dxi = dxi_ref[...]
    mm_full = mm_ref[...]
    qli = qli_ref[...]
    qlj = qlj_ref[...]
    qsj = qsj_ref[...]
    qgj = qgj_ref[...]
    mij_mid = _mid_rows(qlj[0:1] - qsj, qlj[1:M] - qli[0:M - 1],
                        qgj - qlj[M - 1:M], dxi, M)
    mij4 = _bounds(mij_mid)
    mij_last = 2.0 * mij4[-1:] - mij4[-2:-1]
    mij = jnp.concatenate([mij4, mij_last], axis=0)

    e_ij = (jnp.abs(mm_full[1:M + 5] - mij[0:M + 4]) +
            0.5 * jnp.abs(mm_full[1:M + 5] + mij[0:M + 4]))
    f1 = e_ij[2:M + 4]
    f2 = e_ij[0:M + 2]
    f12_ij = f1 + f2
    msk = f12_ij > 1e-09 * gmax
    df = (f1 * mij[1:M + 3] + f2 * mij[2:M + 4]) / jnp.where(msk, f12_ij, 1.0)
    df = jnp.where(msk, df, 0.5 * (mij[3:M + 5] + mij[0:M + 2]))

    slope = mij[2:M + 3]
    y0 = jnp.concatenate([qsj, qli[0:M - 1], qlj[M - 1:M]], axis=0)
    d0 = df[0:M + 1]
    d1 = df[1:M + 2]
    hinv = dxi
    c0 = (d0 + d1 - 2.0 * slope) * hinv * hinv
    c1 = (3.0 * slope - 2.0 * d0 - d1) * hinv
    out_ref[...] = jnp.concatenate([c0, c1, d0, y0], axis=1)


def _sc_eval(ctab, xq, M, dim, Q):
    NC, NS, L = 2, 16, 16
    NW = NC * NS
    CHUNK = 4096
    qpw = Q // NW
    nchunks = qpw // CHUNK
    rows = M + 1

    mesh = plsc.VectorSubcoreMesh(core_axis_name="c", subcore_axis_name="s",
                                  num_cores=NC, num_subcores=NS)

    @functools.partial(
        pl.kernel,
        out_type=jax.ShapeDtypeStruct((dim, Q), jnp.float32),
        mesh=mesh,
        compiler_params=pltpu.CompilerParams(needs_layout_passes=False,
                                             disable_bounds_checks=True),
        scratch_types=[
            pltpu.VMEM((4 * dim, rows), jnp.float32),
            pltpu.VMEM((2, CHUNK), jnp.float32),
            pltpu.VMEM((dim, CHUNK), jnp.float32),
            pltpu.SemaphoreType.DMA,
            pltpu.SemaphoreType.DMA,
            pltpu.SemaphoreType.DMA,
        ],
    )
    def sc_eval(ctabt_hbm, xq_hbm, y_hbm, tab_v, xq_v, y_v, si0, si1, so):
        wid = lax.axis_index("s") * NC + lax.axis_index("c")
        base = wid * qpw
        pltpu.sync_copy(ctabt_hbm, tab_v)
        sin = (si0, si1)
        in_cp = [None, None]
        in_cp[0] = pltpu.async_copy(
            xq_hbm.at[pl.ds(base, CHUNK)], xq_v.at[0], sin[0])
        for c in range(nchunks):
            cur = c % 2
            start = base + c * CHUNK
            if c + 1 < nchunks:
                in_cp[1 - cur] = pltpu.async_copy(
                    xq_hbm.at[pl.ds(start + CHUNK, CHUNK)],
                    xq_v.at[1 - cur], sin[1 - cur])
            in_cp[cur].wait()
            if c > 0:
                for cp in out_cps:
                    cp.wait()

            @plsc.parallel_loop(0, CHUNK // L, unroll=2)
            def body(g):
                xv = xq_v[cur, pl.ds(g * L, L)]
                seg = jnp.clip(xv.astype(jnp.int32), 0, M)
                t = xv - seg.astype(jnp.float32)
                for d in range(dim):
                    k0 = jnp.full((L,), d, jnp.int32)
                    c0 = plsc.load_gather(tab_v, [k0, seg])
                    c1 = plsc.load_gather(tab_v, [k0 + dim, seg])
                    c2 = plsc.load_gather(tab_v, [k0 + 2 * dim, seg])
                    c3 = plsc.load_gather(tab_v, [k0 + 3 * dim, seg])
                    y_v[d, pl.ds(g * L, L)] = ((c0 * t + c1) * t + c2) * t + c3

            out_cps = [
                pltpu.async_copy(y_v.at[d], y_hbm.at[d, pl.ds(start, CHUNK)],
                                 so)
                for d in range(dim)
            ]
        for cp in out_cps:
            cp.wait()

    yt = sc_eval(jnp.transpose(ctab), xq)
    return jnp.transpose(yt)


def kernel(x, q_s, q_l, q_g, xq, i, j):
    M, N, dim = q_l.shape
    Q = xq.shape[0]
    f32 = jnp.float32
    BD = N * dim

    x2 = x.astype(f32).reshape(M + 2, 1)
    qlbd = q_l.reshape(M, BD)
    qsbd = q_s.reshape(1, BD)
    qgbd = q_g.reshape(1, BD)
    qli = lax.dynamic_index_in_dim(q_l, i, axis=1, keepdims=False)
    qlj = lax.dynamic_index_in_dim(q_l, j, axis=1, keepdims=False)
    qsj = lax.dynamic_index_in_dim(q_s, j, axis=0, keepdims=True)
    qgj = lax.dynamic_index_in_dim(q_g, j, axis=0, keepdims=True)

    P = jnp.asarray(np.tile(np.eye(dim, dtype=np.float32), (1, N)))
    PM = jnp.asarray(np.tile(np.eye(dim, dtype=np.float32), (N, 1)) / N)

    full = lambda s: pl.BlockSpec(s, lambda *_: (0,) * len(s))

    mm_bd, mm_full, dxi = pl.pallas_call(
        functools.partial(_prep_kernel, M, N, dim),
        in_specs=[full((M + 2, 1)), full((M, BD)), full((1, BD)),
                  full((1, BD)), full((dim, BD)), full((BD, dim))],
        out_specs=[full((M + 5, BD)), full((M + 5, dim)), full((M + 1, 1))],
        out_shape=[jax.ShapeDtypeStruct((M + 5, BD), f32),
                   jax.ShapeDtypeStruct((M + 5, dim), f32),
                   jax.ShapeDtypeStruct((M + 1, 1), f32)],
    )(x2, qlbd, qsbd, qgbd, P, PM)

    gmax = pl.pallas_call(
        functools.partial(_gmax_kernel, M, N, dim),
        grid=(N // 2,),
        in_specs=[full((M, BD)), full((1, BD)), full((1, BD)),
                  full((M + 5, BD)), full((M + 1, 1)), full((dim, BD))],
        out_specs=full((1, 1)),
        out_shape=jax.ShapeDtypeStruct((1, 1), f32),
        scratch_shapes=[pltpu.VMEM((1, 1), f32)],
        compiler_params=pltpu.CompilerParams(
            dimension_semantics=("arbitrary",)),
    )(qlbd, qsbd, qgbd, mm_bd, dxi, P)

    ctab = pl.pallas_call(
        functools.partial(_table_kernel, M, N, dim),
        in_specs=[full((M + 5, dim)), full((M + 1, 1)), full((1, 1)),
                  full((M, dim)), full((M, dim)), full((1, dim)),
                  full((1, dim))],
        out_specs=full((M + 1, 4 * dim)),
        out_shape=jax.ShapeDtypeStruct((M + 1, 4 * dim), f32),
    )(mm_full, dxi, gmax, qli, qlj, qsj, qgj)

    return _sc_eval(ctab, xq.astype(f32), M, dim, Q)


# speedup vs baseline: 287.6513x; 1.2888x over previous
import functools

import jax
import jax.numpy as jnp
import numpy as np
from jax import lax
from jax.experimental import pallas as pl
from jax.experimental.pallas import tpu as pltpu
from jax.experimental.pallas import tpu_sc as plsc


def _bounds(mid):
    r1 = 2.0 * mid[0:1] - mid[1:2]
    r0 = 2.0 * r1 - mid[0:1]
    rp = 2.0 * mid[-1:] - mid[-2:-1]
    return jnp.concatenate([r0, r1, mid, rp], axis=0)


def _dxi_of(x):
    dx = x[1:, :] - x[:-1, :]
    mask0 = dx == 0.0
    return jnp.where(mask0, 0.0, 1.0 / jnp.where(mask0, 1.0, dx))


def _mid_rows(first, mids, last, dxi, M):
    return jnp.concatenate([
        first * dxi[0:1],
        mids * dxi[1:M],
        last * dxi[M:M + 1],
    ], axis=0)


def _prep_kernel(M, N, dim, x_ref, qlbd_ref, qsbd_ref, qgbd_ref, p_ref,
                 pm_ref, mmbd_ref, mm_ref, dxi_ref):
    f32 = jnp.float32
    dxi = _dxi_of(x_ref[...])
    qlbd = qlbd_ref[...]
    dn = (((1,), (0,)), ((), ()))
    ql_mean = lax.dot_general(qlbd, pm_ref[...], dn, preferred_element_type=f32)
    qs_mean = lax.dot_general(qsbd_ref[...], pm_ref[...], dn,
                              preferred_element_type=f32)
    qg_mean = lax.dot_general(qgbd_ref[...], pm_ref[...], dn,
                              preferred_element_type=f32)
    mm_mid = _mid_rows(ql_mean[0:1] - qs_mean,
                       ql_mean[1:M] - ql_mean[0:M - 1],
                       qg_mean - ql_mean[M - 1:M], dxi, M)
    mm4 = _bounds(mm_mid)
    mm_last = 2.0 * mm4[-1:] - mm4[-2:-1]
    mm_full = jnp.concatenate([mm4, mm_last], axis=0)
    mmbd_ref[...] = lax.dot_general(mm_full, p_ref[...], dn,
                                    preferred_element_type=f32)
    mm_ref[...] = mm_full
    dxi_ref[...] = dxi


def _gmax_kernel(M, N, dim, qlbd_ref, qsbd_ref, qgbd_ref, mmbd_ref, dxi_ref,
                 p_ref, out_ref, acc_ref):
    step = pl.program_id(0)
    BD = N * dim
    f32 = jnp.float32
    dn = (((1,), (0,)), ((), ()))
    qlbd = qlbd_ref[...]
    dxi = dxi_ref[...]
    mmbd = mmbd_ref[...]
    qsbd = qsbd_ref[...]
    qgbd = qgbd_ref[...]
    rows = lax.broadcasted_iota(jnp.int32, (BD, dim), 0)
    cols = lax.broadcasted_iota(jnp.int32, (BD, dim), 1)

    gm = None
    for half in range(2):
        a = step * 2 + half
        psel = (rows == a * dim + cols).astype(f32)
        a_slice = lax.dot_general(qlbd, psel, dn, preferred_element_type=f32)
        a_bd = lax.dot_general(a_slice, p_ref[...], dn,
                               preferred_element_type=f32)
        m_mid = _mid_rows(qlbd[0:1] - qsbd,
                          qlbd[1:M] - a_bd[0:M - 1],
                          qgbd - qlbd[M - 1:M], dxi, M)
        m4 = _bounds(m_mid)
        e = (jnp.abs(mmbd[1:M + 5] - m4) +
             0.5 * jnp.abs(mmbd[1:M + 5] + m4))
        f12 = e[2:M + 4] + e[0:M + 2]
        gm_h = jnp.max(f12).reshape(1, 1)
        gm = gm_h if gm is None else jnp.maximum(gm, gm_h)

    prev = jnp.where(step == 0, -jnp.inf, acc_ref[...])
    acc_ref[...] = jnp.maximum(prev, gm)

    @pl.when(step == pl.num_programs(0) - 1)
    def _():
        out_ref[...] = acc_ref[...]


def _table_kernel(M, N, dim, mm_ref, dxi_ref, gmax_ref, qli_ref, qlj_ref,
                  qsj_ref, qgj_ref, out_ref):
    gmax = gmax_ref[0, 0]
    dxi = dxi_ref[...]
    mm_full = mm_ref[...]
    qli = qli_ref[...]
    qlj = qlj_ref[...]
    qsj = qsj_ref[...]
    qgj = qgj_ref[...]
    mij_mid = _mid_rows(qlj[0:1] - qsj, qlj[1:M] - qli[0:M - 1],
                        qgj - qlj[M - 1:M], dxi, M)
    mij4 = _bounds(mij_mid)
    mij_last = 2.0 * mij4[-1:] - mij4[-2:-1]
    mij = jnp.concatenate([mij4, mij_last], axis=0)

    e_ij = (jnp.abs(mm_full[1:M + 5] - mij[0:M + 4]) +
            0.5 * jnp.abs(mm_full[1:M + 5] + mij[0:M + 4]))
    f1 = e_ij[2:M + 4]
    f2 = e_ij[0:M + 2]
    f12_ij = f1 + f2
    msk = f12_ij > 1e-09 * gmax
    df = (f1 * mij[1:M + 3] + f2 * mij[2:M + 4]) / jnp.where(msk, f12_ij, 1.0)
    df = jnp.where(msk, df, 0.5 * (mij[3:M + 5] + mij[0:M + 2]))

    slope = mij[2:M + 3]
    y0 = jnp.concatenate([qsj, qli[0:M - 1], qlj[M - 1:M]], axis=0)
    d0 = df[0:M + 1]
    d1 = df[1:M + 2]
    hinv = dxi
    c0 = (d0 + d1 - 2.0 * slope) * hinv * hinv
    c1 = (3.0 * slope - 2.0 * d0 - d1) * hinv
    out_ref[...] = jnp.concatenate([c0, c1, d0, y0], axis=1)


def _sc_eval(ctab, xq, M, dim, Q):
    NC, NS, L = 2, 16, 16
    NW = NC * NS
    CHUNK = 2048
    qpw = Q // NW
    nchunks = qpw // CHUNK
    rows = M + 1

    mesh = plsc.VectorSubcoreMesh(core_axis_name="c", subcore_axis_name="s",
                                  num_cores=NC, num_subcores=NS)

    @functools.partial(
        pl.kernel,
        out_type=jax.ShapeDtypeStruct((dim, Q), jnp.float32),
        mesh=mesh,
        compiler_params=pltpu.CompilerParams(needs_layout_passes=False,
                                             disable_bounds_checks=True),
        scratch_types=[
            pltpu.VMEM((rows, 4 * dim), jnp.float32),
            pltpu.VMEM((CHUNK,), jnp.float32),
            pltpu.VMEM((dim, CHUNK), jnp.float32),
            pltpu.VMEM((8 * L * L,), jnp.float32),
            pltpu.SemaphoreType.DMA,
        ],
    )
    def sc_eval(ctab_hbm, xq_hbm, y_hbm, tab_v, xq_v, y_v, st_v, so):
        wid = lax.axis_index("s") * NC + lax.axis_index("c")
        base = wid * qpw
        pltpu.sync_copy(ctab_hbm, tab_v)
        iota = lax.iota(jnp.int32, L)
        for c in range(nchunks):
            start = base + c * CHUNK
            pltpu.sync_copy(xq_hbm.at[pl.ds(start, CHUNK)], xq_v)
            if c > 0:
                for cp in out_cps:
                    cp.wait()

            @plsc.parallel_loop(0, CHUNK // L, unroll=1)
            def body(g):
                xv = xq_v[pl.ds(g * L, L)]
                seg = jnp.clip(xv.astype(jnp.int32), 0, M)
                t16 = xv - seg.astype(jnp.float32)
                sbase = (g & 7) * (L * L)
                for l in range(L):
                    s = seg[l]
                    t = jnp.broadcast_to(t16[l], (L,))
                    c0 = tab_v[s, 0:dim]
                    c1 = tab_v[s, dim:2 * dim]
                    c2 = tab_v[s, 2 * dim:3 * dim]
                    c3 = tab_v[s, 3 * dim:4 * dim]
                    plsc.store_scatter(
                        st_v, [sbase + l * L + ((iota + l) & (L - 1))],
                        ((c0 * t + c1) * t + c2) * t + c3)
                for d in range(dim):
                    idx = sbase + iota * L + ((iota + d) & (L - 1))
                    y_v[d, pl.ds(g * L, L)] = plsc.load_gather(st_v, [idx])

            out_cps = [
                pltpu.async_copy(y_v.at[d], y_hbm.at[d, pl.ds(start, CHUNK)],
                                 so)
                for d in range(dim)
            ]
        for cp in out_cps:
            cp.wait()

    yt = sc_eval(ctab, xq)
    return jnp.transpose(yt)


def kernel(x, q_s, q_l, q_g, xq, i, j):
    M, N, dim = q_l.shape
    Q = xq.shape[0]
    f32 = jnp.float32
    BD = N * dim

    x2 = x.astype(f32).reshape(M + 2, 1)
    qlbd = q_l.reshape(M, BD)
    qsbd = q_s.reshape(1, BD)
    qgbd = q_g.reshape(1, BD)
    qli = lax.dynamic_index_in_dim(q_l, i, axis=1, keepdims=False)
    qlj = lax.dynamic_index_in_dim(q_l, j, axis=1, keepdims=False)
    qsj = lax.dynamic_index_in_dim(q_s, j, axis=0, keepdims=True)
    qgj = lax.dynamic_index_in_dim(q_g, j, axis=0, keepdims=True)

    P = jnp.asarray(np.tile(np.eye(dim, dtype=np.float32), (1, N)))
    PM = jnp.asarray(np.tile(np.eye(dim, dtype=np.float32), (N, 1)) / N)

    full = lambda s: pl.BlockSpec(s, lambda *_: (0,) * len(s))

    mm_bd, mm_full, dxi = pl.pallas_call(
        functools.partial(_prep_kernel, M, N, dim),
        in_specs=[full((M + 2, 1)), full((M, BD)), full((1, BD)),
                  full((1, BD)), full((dim, BD)), full((BD, dim))],
        out_specs=[full((M + 5, BD)), full((M + 5, dim)), full((M + 1, 1))],
        out_shape=[jax.ShapeDtypeStruct((M + 5, BD), f32),
                   jax.ShapeDtypeStruct((M + 5, dim), f32),
                   jax.ShapeDtypeStruct((M + 1, 1), f32)],
    )(x2, qlbd, qsbd, qgbd, P, PM)

    gmax = pl.pallas_call(
        functools.partial(_gmax_kernel, M, N, dim),
        grid=(N // 2,),
        in_specs=[full((M, BD)), full((1, BD)), full((1, BD)),
                  full((M + 5, BD)), full((M + 1, 1)), full((dim, BD))],
        out_specs=full((1, 1)),
        out_shape=jax.ShapeDtypeStruct((1, 1), f32),
        scratch_shapes=[pltpu.VMEM((1, 1), f32)],
        compiler_params=pltpu.CompilerParams(
            dimension_semantics=("arbitrary",)),
    )(qlbd, qsbd, qgbd, mm_bd, dxi, P)

    ctab = pl.pallas_call(
        functools.partial(_table_kernel, M, N, dim),
        in_specs=[full((M + 5, dim)), full((M + 1, 1)), full((1, 1)),
                  full((M, dim)), full((M, dim)), full((1, dim)),
                  full((1, dim))],
        out_specs=full((M + 1, 4 * dim)),
        out_shape=jax.ShapeDtypeStruct((M + 1, 4 * dim), f32),
    )(mm_full, dxi, gmax, qli, qlj, qsj, qgj)

    return _sc_eval(ctab, xq.astype(f32), M, dim, Q)


# speedup vs baseline: 296.7628x; 1.0317x over previous
import functools

import jax
import jax.numpy as jnp
import numpy as np
from jax import lax
from jax.experimental import pallas as pl
from jax.experimental.pallas import tpu as pltpu
from jax.experimental.pallas import tpu_sc as plsc


def _bounds(mid):
    r1 = 2.0 * mid[0:1] - mid[1:2]
    r0 = 2.0 * r1 - mid[0:1]
    rp = 2.0 * mid[-1:] - mid[-2:-1]
    return jnp.concatenate([r0, r1, mid, rp], axis=0)


def _dxi_of(x):
    dx = x[1:, :] - x[:-1, :]
    mask0 = dx == 0.0
    return jnp.where(mask0, 0.0, 1.0 / jnp.where(mask0, 1.0, dx))


def _mid_rows(first, mids, last, dxi, M):
    return jnp.concatenate([
        first * dxi[0:1],
        mids * dxi[1:M],
        last * dxi[M:M + 1],
    ], axis=0)


def _prep_kernel(M, N, dim, x_ref, qlbd_ref, qsbd_ref, qgbd_ref, p_ref,
                 pm_ref, mmbd_ref, mm_ref, dxi_ref):
    f32 = jnp.float32
    dxi = _dxi_of(x_ref[...])
    qlbd = qlbd_ref[...]
    dn = (((1,), (0,)), ((), ()))
    ql_mean = lax.dot_general(qlbd, pm_ref[...], dn, preferred_element_type=f32)
    qs_mean = lax.dot_general(qsbd_ref[...], pm_ref[...], dn,
                              preferred_element_type=f32)
    qg_mean = lax.dot_general(qgbd_ref[...], pm_ref[...], dn,
                              preferred_element_type=f32)
    mm_mid = _mid_rows(ql_mean[0:1] - qs_mean,
                       ql_mean[1:M] - ql_mean[0:M - 1],
                       qg_mean - ql_mean[M - 1:M], dxi, M)
    mm4 = _bounds(mm_mid)
    mm_last = 2.0 * mm4[-1:] - mm4[-2:-1]
    mm_full = jnp.concatenate([mm4, mm_last], axis=0)
    mmbd_ref[...] = lax.dot_general(mm_full, p_ref[...], dn,
                                    preferred_element_type=f32)
    mm_ref[...] = mm_full
    dxi_ref[...] = dxi


def _gmax_kernel(M, N, dim, qlbd_ref, qsbd_ref, qgbd_ref, mmbd_ref, dxi_ref,
                 p_ref, out_ref, acc_ref):
    step = pl.program_id(0)
    BD = N * dim
    f32 = jnp.float32
    dn = (((1,), (0,)), ((), ()))
    qlbd = qlbd_ref[...]
    dxi = dxi_ref[...]
    mmbd = mmbd_ref[...]
    qsbd = qsbd_ref[...]
    qgbd = qgbd_ref[...]
    rows = lax.broadcasted_iota(jnp.int32, (BD, dim), 0)
    cols = lax.broadcasted_iota(jnp.int32, (BD, dim), 1)

    gm = None
    for half in range(4):
        a = step * 4 + half
        psel = (rows == a * dim + cols).astype(f32)
        a_slice = lax.dot_general(qlbd, psel, dn, preferred_element_type=f32)
        a_bd = lax.dot_general(a_slice, p_ref[...], dn,
                               preferred_element_type=f32)
        m_mid = _mid_rows(qlbd[0:1] - qsbd,
                          qlbd[1:M] - a_bd[0:M - 1],
                          qgbd - qlbd[M - 1:M], dxi, M)
        m4 = _bounds(m_mid)
        e = (jnp.abs(mmbd[1:M + 5] - m4) +
             0.5 * jnp.abs(mmbd[1:M + 5] + m4))
        f12 = e[2:M + 4] + e[0:M + 2]
        gm_h = jnp.max(f12).reshape(1, 1)
        gm = gm_h if gm is None else jnp.maximum(gm, gm_h)

    prev = jnp.where(step == 0, -jnp.inf, acc_ref[...])
    acc_ref[...] = jnp.maximum(prev, gm)

    @pl.when(step == pl.num_programs(0) - 1)
    def _():
        out_ref[...] = acc_ref[...]


def _table_kernel(M, N, dim, mm_ref, dxi_ref, gmax_ref, qli_ref, qlj_ref,
                  qsj_ref, qgj_ref, out_ref):
    gmax = gmax_ref[0, 0]
    dxi = dxi_ref[...]
    mm_full = mm_ref[...]
    qli = qli_ref[...]
    qlj = qlj_ref[...]
    qsj = qsj_ref[...]
    qgj = qgj_ref[...]
    mij_mid = _mid_rows(qlj[0:1] - qsj, qlj[1:M] - qli[0:M - 1],
                        qgj - qlj[M - 1:M], dxi, M)
    mij4 = _bounds(mij_mid)
    mij_last = 2.0 * mij4[-1:] - mij4[-2:-1]
    mij = jnp.concatenate([mij4, mij_last], axis=0)

    e_ij = (jnp.abs(mm_full[1:M + 5] - mij[0:M + 4]) +
            0.5 * jnp.abs(mm_full[1:M + 5] + mij[0:M + 4]))
    f1 = e_ij[2:M + 4]
    f2 = e_ij[0:M + 2]
    f12_ij = f1 + f2
    msk = f12_ij > 1e-09 * gmax
    df = (f1 * mij[1:M + 3] + f2 * mij[2:M + 4]) / jnp.where(msk, f12_ij, 1.0)
    df = jnp.where(msk, df, 0.5 * (mij[3:M + 5] + mij[0:M + 2]))

    slope = mij[2:M + 3]
    y0 = jnp.concatenate([qsj, qli[0:M - 1], qlj[M - 1:M]], axis=0)
    d0 = df[0:M + 1]
    d1 = df[1:M + 2]
    hinv = dxi
    c0 = (d0 + d1 - 2.0 * slope) * hinv * hinv
    c1 = (3.0 * slope - 2.0 * d0 - d1) * hinv
    out_ref[...] = jnp.concatenate([c0, c1, d0, y0], axis=1)


def _sc_eval(ctab, xq, M, dim, Q):
    NC, NS, L = 2, 16, 16
    NW = NC * NS
    CHUNK = 2048
    qpw = Q // NW
    nchunks = qpw // CHUNK
    rows = M + 1

    mesh = plsc.VectorSubcoreMesh(core_axis_name="c", subcore_axis_name="s",
                                  num_cores=NC, num_subcores=NS)

    @functools.partial(
        pl.kernel,
        out_type=jax.ShapeDtypeStruct((dim, Q), jnp.float32),
        mesh=mesh,
        compiler_params=pltpu.CompilerParams(needs_layout_passes=False,
                                             disable_bounds_checks=True),
        scratch_types=[
            pltpu.VMEM((rows, 4 * dim), jnp.float32),
            pltpu.VMEM((CHUNK,), jnp.float32),
            pltpu.VMEM((dim, CHUNK), jnp.float32),
            pltpu.VMEM((8 * L * L,), jnp.float32),
            pltpu.SemaphoreType.DMA,
        ],
    )
    def sc_eval(ctab_hbm, xq_hbm, y_hbm, tab_v, xq_v, y_v, st_v, so):
        wid = lax.axis_index("s") * NC + lax.axis_index("c")
        base = wid * qpw
        pltpu.sync_copy(ctab_hbm, tab_v)
        iota = lax.iota(jnp.int32, L)
        for c in range(nchunks):
            start = base + c * CHUNK
            pltpu.sync_copy(xq_hbm.at[pl.ds(start, CHUNK)], xq_v)
            if c > 0:
                for cp in out_cps:
                    cp.wait()

            @plsc.parallel_loop(0, CHUNK // L, unroll=1)
            def body(g):
                xv = xq_v[pl.ds(g * L, L)]
                seg = jnp.clip(xv.astype(jnp.int32), 0, M)
                t16 = xv - seg.astype(jnp.float32)
                sbase = (g & 7) * (L * L)
                for l in range(L):
                    s = seg[l]
                    t = jnp.broadcast_to(t16[l], (L,))
                    c0 = tab_v[s, 0:dim]
                    c1 = tab_v[s, dim:2 * dim]
                    c2 = tab_v[s, 2 * dim:3 * dim]
                    c3 = tab_v[s, 3 * dim:4 * dim]
                    plsc.store_scatter(
                        st_v, [sbase + l * L + ((iota + l) & (L - 1))],
                        ((c0 * t + c1) * t + c2) * t + c3)
                for d in range(dim):
                    idx = sbase + iota * L + ((iota + d) & (L - 1))
                    y_v[d, pl.ds(g * L, L)] = plsc.load_gather(st_v, [idx])

            out_cps = [
                pltpu.async_copy(y_v.at[d], y_hbm.at[d, pl.ds(start, CHUNK)],
                                 so)
                for d in range(dim)
            ]
        for cp in out_cps:
            cp.wait()

    yt = sc_eval(ctab, xq)
    return jnp.transpose(yt)


def kernel(x, q_s, q_l, q_g, xq, i, j):
    M, N, dim = q_l.shape
    Q = xq.shape[0]
    f32 = jnp.float32
    BD = N * dim

    x2 = x.astype(f32).reshape(M + 2, 1)
    qlbd = q_l.reshape(M, BD)
    qsbd = q_s.reshape(1, BD)
    qgbd = q_g.reshape(1, BD)
    qli = lax.dynamic_index_in_dim(q_l, i, axis=1, keepdims=False)
    qlj = lax.dynamic_index_in_dim(q_l, j, axis=1, keepdims=False)
    qsj = lax.dynamic_index_in_dim(q_s, j, axis=0, keepdims=True)
    qgj = lax.dynamic_index_in_dim(q_g, j, axis=0, keepdims=True)

    P = jnp.asarray(np.tile(np.eye(dim, dtype=np.float32), (1, N)))
    PM = jnp.asarray(np.tile(np.eye(dim, dtype=np.float32), (N, 1)) / N)

    full = lambda s: pl.BlockSpec(s, lambda *_: (0,) * len(s))

    mm_bd, mm_full, dxi = pl.pallas_call(
        functools.partial(_prep_kernel, M, N, dim),
        in_specs=[full((M + 2, 1)), full((M, BD)), full((1, BD)),
                  full((1, BD)), full((dim, BD)), full((BD, dim))],
        out_specs=[full((M + 5, BD)), full((M + 5, dim)), full((M + 1, 1))],
        out_shape=[jax.ShapeDtypeStruct((M + 5, BD), f32),
                   jax.ShapeDtypeStruct((M + 5, dim), f32),
                   jax.ShapeDtypeStruct((M + 1, 1), f32)],
    )(x2, qlbd, qsbd, qgbd, P, PM)

    gmax = pl.pallas_call(
        functools.partial(_gmax_kernel, M, N, dim),
        grid=(N // 4,),
        in_specs=[full((M, BD)), full((1, BD)), full((1, BD)),
                  full((M + 5, BD)), full((M + 1, 1)), full((dim, BD))],
        out_specs=full((1, 1)),
        out_shape=jax.ShapeDtypeStruct((1, 1), f32),
        scratch_shapes=[pltpu.VMEM((1, 1), f32)],
        compiler_params=pltpu.CompilerParams(
            dimension_semantics=("arbitrary",)),
    )(qlbd, qsbd, qgbd, mm_bd, dxi, P)

    ctab = pl.pallas_call(
        functools.partial(_table_kernel, M, N, dim),
        in_specs=[full((M + 5, dim)), full((M + 1, 1)), full((1, 1)),
                  full((M, dim)), full((M, dim)), full((1, dim)),
                  full((1, dim))],
        out_specs=full((M + 1, 4 * dim)),
        out_shape=jax.ShapeDtypeStruct((M + 1, 4 * dim), f32),
    )(mm_full, dxi, gmax, qli, qlj, qsj, qgj)

    return _sc_eval(ctab, xq.astype(f32), M, dim, Q)
